# baseline (speedup 1.0000x reference)
import functools

import jax
import jax.numpy as jnp
from jax import lax
from jax.experimental import pallas as pl
from jax.experimental.pallas import tpu as pltpu
from jax.experimental.pallas import tpu_sc as plsc

_ROWS = 16384
_D = 1024

_SC_ROWS = 4096
_TC_ROWS = _ROWS - _SC_ROWS

_L = 16
_NW = 32
_RPW = _SC_ROWS // _NW
_C = 32
_NCHUNK = _RPW // _C

_mesh = plsc.VectorSubcoreMesh(core_axis_name="c", subcore_axis_name="s")


@functools.partial(
    pl.kernel,
    out_type=jax.ShapeDtypeStruct((_NW, 2, _L), jnp.float32),
    mesh=_mesh,
    scratch_types=[
        pltpu.VMEM((_C, _D), jnp.float32),
        pltpu.VMEM((_C, _D), jnp.float32),
        pltpu.VMEM((_RPW + _L,), jnp.float32),
        pltpu.VMEM((_RPW + _L,), jnp.float32),
        pltpu.VMEM((2, _L), jnp.float32),
        pltpu.SemaphoreType.DMA,
        pltpu.SemaphoreType.DMA,
    ],
    compiler_params=pltpu.CompilerParams(needs_layout_passes=False),
)
def _sc_partials(x_hbm, yt_hbm, yp_hbm, out_hbm,
                 buf0, buf1, yt_v, yp_v, res_v, sem0, sem1):
    wid = lax.axis_index("s") * 2 + lax.axis_index("c")
    base = wid * _RPW

    pltpu.sync_copy(yt_hbm.at[pl.ds(base, _RPW)], yt_v.at[pl.ds(0, _RPW)])
    pltpu.sync_copy(yp_hbm.at[pl.ds(base, _RPW)], yp_v.at[pl.ds(0, _RPW)])

    bufs = (buf0, buf1)
    sems = (sem0, sem1)
    copies = [pltpu.async_copy(x_hbm.at[pl.ds(base, _C)], buf0, sem0), None]

    num = jnp.float32(0.0)
    den = jnp.float32(0.0)
    for ch in range(_NCHUNK):
        if ch + 1 < _NCHUNK:
            nxt = (ch + 1) % 2
            copies[nxt] = pltpu.async_copy(
                x_hbm.at[pl.ds(base + (ch + 1) * _C, _C)], bufs[nxt], sems[nxt])
        copies[ch % 2].wait()
        buf = bufs[ch % 2]

        def _row(r, carry, buf=buf, ch=ch):
            num, den = carry
            vals = [buf[r, pl.ds(k * _L, _L)] for k in range(_D // _L)]
            while len(vals) > 1:
                nxt_vals = [vals[i] + vals[i + 1] for i in range(0, len(vals) - 1, 2)]
                if len(vals) % 2:
                    nxt_vals.append(vals[-1])
                vals = nxt_vals
            s = jnp.sum(vals[0])
            m = jnp.where(s > 0.0, jnp.float32(1.0), jnp.float32(0.0))
            yt = yt_v[pl.ds(ch * _C + r, _L)][0]
            yp = yp_v[pl.ds(ch * _C + r, _L)][0]
            c = jnp.where(
                (yp > 0.5) & (yt == 1.0) | (yp <= 0.5) & (yt == 0.0),
                jnp.float32(1.0), jnp.float32(0.0))
            return num + c * m, den + m

        num, den = lax.fori_loop(0, _C, _row, (num, den), unroll=False)

    res_v[0, pl.ds(0, _L)] = jnp.full((_L,), num, jnp.float32)
    res_v[1, pl.ds(0, _L)] = jnp.full((_L,), den, jnp.float32)
    pltpu.sync_copy(res_v, out_hbm.at[wid])


_BLK = 2048
_GRID = _TC_ROWS // _BLK


def _tc_body(x_ref, yt_ref, yp_ref, out_ref, acc_ref):
    i = pl.program_id(0)

    @pl.when(i == 0)
    def _init():
        acc_ref[0] = 0.0
        acc_ref[1] = 0.0

    rowsum = jnp.sum(x_ref[...], axis=1)
    mask = (rowsum > 0.0).astype(jnp.float32)
    yt = yt_ref[0, 0, :]
    yp = yp_ref[0, 0, :]
    correct = jnp.where(
        (yp > 0.5) & (yt == 1.0) | (yp <= 0.5) & (yt == 0.0), 1.0, 0.0)
    acc_ref[0] += jnp.sum(correct * mask)
    acc_ref[1] += jnp.sum(mask)

    @pl.when(i == _GRID - 1)
    def _fin():
        out_ref[...] = jnp.stack([acc_ref[0], acc_ref[1]]).reshape(1, 2)


def _combine_body(sc_ref, tc_ref, out_ref):
    num = jnp.sum(sc_ref[:, 0, :]) / _L + tc_ref[0, 0]
    den = jnp.sum(sc_ref[:, 1, :]) / _L + tc_ref[0, 1]
    out_ref[...] = jnp.full((1, 1), num / den, jnp.float32)


def kernel(input_data, y_true, y_pred):
    x = input_data.reshape(_ROWS, _D)
    yt = y_true.reshape(_ROWS)
    yp = y_pred.reshape(_ROWS)

    sc_parts = _sc_partials(x[_TC_ROWS:], yt[_TC_ROWS:], yp[_TC_ROWS:])

    tc_parts = pl.pallas_call(
        _tc_body,
        grid=(_GRID,),
        in_specs=[
            pl.BlockSpec((_BLK, _D), lambda i: (i, 0)),
            pl.BlockSpec((1, 1, _BLK), lambda i: (i, 0, 0)),
            pl.BlockSpec((1, 1, _BLK), lambda i: (i, 0, 0)),
        ],
        out_specs=pl.BlockSpec((1, 2), lambda i: (0, 0)),
        out_shape=jax.ShapeDtypeStruct((1, 2), jnp.float32),
        scratch_shapes=[pltpu.SMEM((2,), jnp.float32)],
    )(x[:_TC_ROWS],
      yt[:_TC_ROWS].reshape(_GRID, 1, _BLK),
      yp[:_TC_ROWS].reshape(_GRID, 1, _BLK))

    out = pl.pallas_call(
        _combine_body,
        out_shape=jax.ShapeDtypeStruct((1, 1), jnp.float32),
    )(sc_parts, tc_parts)
    return out[0, 0]


# speedup vs baseline: 2.1959x; 2.1959x over previous
import functools

import jax
import jax.numpy as jnp
from jax import lax
from jax.experimental import pallas as pl
from jax.experimental.pallas import tpu as pltpu
from jax.experimental.pallas import tpu_sc as plsc

_ROWS = 16384
_D = 1024

_SC_ROWS = 4096
_TC_ROWS = _ROWS - _SC_ROWS

_L = 16
_NW = 32
_RPW = _SC_ROWS // _NW
_C = 32
_NCHUNK = _RPW // _C

_mesh = plsc.VectorSubcoreMesh(core_axis_name="c", subcore_axis_name="s")


@functools.partial(
    pl.kernel,
    out_type=jax.ShapeDtypeStruct((_NW, 2, _L), jnp.float32),
    mesh=_mesh,
    scratch_types=[
        pltpu.VMEM((_C, _D), jnp.float32),
        pltpu.VMEM((_C, _D), jnp.float32),
        pltpu.VMEM((_RPW + _L,), jnp.float32),
        pltpu.VMEM((_RPW + _L,), jnp.float32),
        pltpu.VMEM((2, _L), jnp.float32),
        pltpu.SemaphoreType.DMA,
        pltpu.SemaphoreType.DMA,
    ],
    compiler_params=pltpu.CompilerParams(needs_layout_passes=False),
)
def _sc_partials(x_hbm, yt_hbm, yp_hbm, out_hbm,
                 buf0, buf1, yt_v, yp_v, res_v, sem0, sem1):
    wid = lax.axis_index("s") * 2 + lax.axis_index("c")
    base = _TC_ROWS + wid * _RPW

    pltpu.sync_copy(yt_hbm.at[pl.ds(base, _RPW)], yt_v.at[pl.ds(0, _RPW)])
    pltpu.sync_copy(yp_hbm.at[pl.ds(base, _RPW)], yp_v.at[pl.ds(0, _RPW)])

    bufs = (buf0, buf1)
    sems = (sem0, sem1)
    copies = [pltpu.async_copy(x_hbm.at[pl.ds(base, _C)], buf0, sem0), None]

    num = jnp.float32(0.0)
    den = jnp.float32(0.0)
    for ch in range(_NCHUNK):
        if ch + 1 < _NCHUNK:
            nxt = (ch + 1) % 2
            copies[nxt] = pltpu.async_copy(
                x_hbm.at[pl.ds(base + (ch + 1) * _C, _C)], bufs[nxt], sems[nxt])
        copies[ch % 2].wait()
        buf = bufs[ch % 2]

        def _row(r, carry, buf=buf, ch=ch):
            num, den = carry
            vals = [buf[r, pl.ds(k * _L, _L)] for k in range(_D // _L)]
            while len(vals) > 1:
                nxt_vals = [vals[i] + vals[i + 1] for i in range(0, len(vals) - 1, 2)]
                if len(vals) % 2:
                    nxt_vals.append(vals[-1])
                vals = nxt_vals
            s = jnp.sum(vals[0])
            m = jnp.where(s > 0.0, jnp.float32(1.0), jnp.float32(0.0))
            yt = yt_v[pl.ds(ch * _C + r, _L)][0]
            yp = yp_v[pl.ds(ch * _C + r, _L)][0]
            c = jnp.where(
                (yp > 0.5) & (yt == 1.0) | (yp <= 0.5) & (yt == 0.0),
                jnp.float32(1.0), jnp.float32(0.0))
            return num + c * m, den + m

        num, den = lax.fori_loop(0, _C, _row, (num, den), unroll=False)

    res_v[0, pl.ds(0, _L)] = jnp.full((_L,), num, jnp.float32)
    res_v[1, pl.ds(0, _L)] = jnp.full((_L,), den, jnp.float32)
    pltpu.sync_copy(res_v, out_hbm.at[wid])


_BLK = 2048
_GRID = _TC_ROWS // _BLK


def _tc_body(x_ref, yt_ref, yp_ref, out_ref, acc_ref):
    i = pl.program_id(0)

    @pl.when(i == 0)
    def _init():
        acc_ref[0] = 0.0
        acc_ref[1] = 0.0

    rowsum = jnp.sum(x_ref[...], axis=1)
    mask = (rowsum > 0.0).astype(jnp.float32)
    yt = yt_ref[0, 0, :]
    yp = yp_ref[0, 0, :]
    correct = jnp.where(
        (yp > 0.5) & (yt == 1.0) | (yp <= 0.5) & (yt == 0.0), 1.0, 0.0)
    acc_ref[0] += jnp.sum(correct * mask)
    acc_ref[1] += jnp.sum(mask)

    @pl.when(i == _GRID - 1)
    def _fin():
        out_ref[...] = jnp.stack([acc_ref[0], acc_ref[1]]).reshape(1, 2)


def _combine_body(sc_ref, tc_ref, out_ref):
    num = jnp.sum(sc_ref[:, 0, :]) / _L + tc_ref[0, 0]
    den = jnp.sum(sc_ref[:, 1, :]) / _L + tc_ref[0, 1]
    out_ref[...] = jnp.full((1, 1), num / den, jnp.float32)


def kernel(input_data, y_true, y_pred):
    x = input_data.reshape(_ROWS, _D)
    yt = y_true.reshape(_ROWS)
    yp = y_pred.reshape(_ROWS)

    sc_parts = _sc_partials(x, yt, yp)

    tc_parts = pl.pallas_call(
        _tc_body,
        grid=(_GRID,),
        in_specs=[
            pl.BlockSpec((_BLK, _D), lambda i: (i, 0)),
            pl.BlockSpec((1, 1, _BLK), lambda i: (i, 0, 0)),
            pl.BlockSpec((1, 1, _BLK), lambda i: (i, 0, 0)),
        ],
        out_specs=pl.BlockSpec((1, 2), lambda i: (0, 0)),
        out_shape=jax.ShapeDtypeStruct((1, 2), jnp.float32),
        scratch_shapes=[pltpu.SMEM((2,), jnp.float32)],
    )(x,
      yt.reshape(_ROWS // _BLK, 1, _BLK),
      yp.reshape(_ROWS // _BLK, 1, _BLK))

    out = pl.pallas_call(
        _combine_body,
        out_shape=jax.ShapeDtypeStruct((1, 1), jnp.float32),
    )(sc_parts, tc_parts)
    return out[0, 0]


# speedup vs baseline: 3.7653x; 1.7147x over previous
import jax
import jax.numpy as jnp
from jax.experimental import pallas as pl
from jax.experimental.pallas import tpu as pltpu

_ROWS = 16384
_D = 1024
_BLK = 2048
_GRID = _ROWS // _BLK


def _body(x_ref, yt_ref, yp_ref, out_ref, acc_ref):
    i = pl.program_id(0)

    @pl.when(i == 0)
    def _init():
        acc_ref[0] = 0.0
        acc_ref[1] = 0.0

    rowsum = jnp.sum(x_ref[...], axis=1)
    mask = (rowsum > 0.0).astype(jnp.float32)
    yt = yt_ref[0, 0, :]
    yp = yp_ref[0, 0, :]
    correct = jnp.where(
        (yp > 0.5) & (yt == 1.0) | (yp <= 0.5) & (yt == 0.0), 1.0, 0.0)
    acc_ref[0] += jnp.sum(correct * mask)
    acc_ref[1] += jnp.sum(mask)

    @pl.when(i == _GRID - 1)
    def _fin():
        out_ref[...] = jnp.full((1, 1), acc_ref[0] / acc_ref[1], jnp.float32)


def kernel(input_data, y_true, y_pred):
    x = input_data.reshape(_ROWS, _D)
    yt = y_true.reshape(_GRID, 1, _BLK)
    yp = y_pred.reshape(_GRID, 1, _BLK)
    out = pl.pallas_call(
        _body,
        grid=(_GRID,),
        in_specs=[
            pl.BlockSpec((_BLK, _D), lambda i: (i, 0)),
            pl.BlockSpec((1, 1, _BLK), lambda i: (i, 0, 0)),
            pl.BlockSpec((1, 1, _BLK), lambda i: (i, 0, 0)),
        ],
        out_specs=pl.BlockSpec((1, 1), lambda i: (0, 0)),
        out_shape=jax.ShapeDtypeStruct((1, 1), jnp.float32),
        scratch_shapes=[pltpu.SMEM((2,), jnp.float32)],
    )(x, yt, yp)
    return out[0, 0]
